# baseline (speedup 1.0000x reference)
import dataclasses
import functools

import jax
import jax.numpy as jnp
from jax import lax
from jax.experimental import pallas as pl
from jax.experimental.pallas import tpu as pltpu
from jax.experimental.pallas import tpu_sc as plsc

N = 320000
C = 128
S = 10000
S_PAD = 10240
EPS = 1e-5

NC = 2
NS = 16
NW = NC * NS
ROWS_W = N // NW
BLK = 80
NBLK = ROWS_W // BLK
SROWS = S_PAD // NS

BS = 4000
RB = 2048

_MESH = plsc.VectorSubcoreMesh(core_axis_name="c", subcore_axis_name="s")

_CP = pltpu.CompilerParams()
if "needs_layout_passes" in pltpu.CompilerParams.__dataclass_fields__:
  _CP = dataclasses.replace(_CP, needs_layout_passes=False)


def _sc_segsum(x, batch_i32, zrow, i80):
  @functools.partial(
      pl.kernel,
      out_type=(
          jax.ShapeDtypeStruct((NC * S_PAD, C), jnp.float32),
          jax.ShapeDtypeStruct((NC * 80, C), jnp.float32),
      ),
      mesh=_MESH,
      compiler_params=_CP,
      scratch_types=[
          pltpu.VMEM((BLK, C), jnp.float32),
          pltpu.VMEM((BLK, C), jnp.float32),
          pltpu.VMEM((BLK, C), jnp.float32),
          pltpu.VMEM((BLK,), jnp.int32),
          pltpu.VMEM((BLK,), jnp.int32),
          pltpu.VMEM((BLK,), jnp.int32),
          pltpu.VMEM((80, C), jnp.float32),
          pltpu.VMEM((80,), jnp.int32),
          pltpu.VMEM_SHARED((S_PAD, C), jnp.float32),
          pltpu.VMEM_SHARED((80, C), jnp.float32),
          pltpu.SemaphoreType.DMA,
          pltpu.SemaphoreType.DMA,
          pltpu.SemaphoreType.DMA,
          pltpu.SemaphoreType.DMA,
          pltpu.SemaphoreType.DMA,
          pltpu.SemaphoreType.DMA,
          pltpu.SemaphoreType.DMA,
          pltpu.SemaphoreType.DMA,
      ],
  )
  def k(x_hbm, b_hbm, zrow_hbm, i80_hbm, out_hbm, cnt_hbm,
        xb0, xb1, xb2, ib0, ib1, ib2, hist, i80buf, acc, shist,
        sx0, sx1, sx2, si0, si1, si2, ssc0, ssc1):
    xbs = (xb0, xb1, xb2)
    ibs = (ib0, ib1, ib2)
    sxs = (sx0, sx1, sx2)
    sis = (si0, si1, si2)
    sscs = (ssc0, ssc1)
    cid = lax.axis_index("c")
    sid = lax.axis_index("s")
    wid = cid * NS + sid

    pltpu.sync_copy(zrow_hbm, acc.at[pl.ds(sid * SROWS, SROWS)])
    pltpu.sync_copy(zrow_hbm.at[pl.ds(0, 80)], hist)
    pltpu.sync_copy(i80_hbm, i80buf)

    @pl.when(sid == 0)
    def _():
      pltpu.sync_copy(zrow_hbm.at[pl.ds(0, 80)], shist)

    plsc.subcore_barrier()

    base_w = wid * ROWS_W
    io = lax.iota(jnp.int32, 16)
    pidx = jnp.maximum(io - 1, 0)
    nidx = jnp.minimum(io + 1, 15)

    def take16(v, idx):
      return lax.gather(
          v,
          idx[:, None],
          lax.GatherDimensionNumbers(
              offset_dims=(),
              collapsed_slice_dims=(0,),
              start_index_map=(0,),
          ),
          (1,),
          mode=lax.GatherScatterMode.PROMISE_IN_BOUNDS,
      )

    def hist_update(ib):
      for kk in range(BLK // 16):
        v = ib[pl.ds(16 * kk, 16)]
        prev = take16(v, pidx)
        nxt = take16(v, nidx)
        start_m = (v != prev) | (io == 0)
        s = jnp.where(start_m, io, 0)
        fm = plsc.cummax(s)
        runlen = (io - fm + 1).astype(jnp.float32)
        endm = (v != nxt) | (io == 15)
        plsc.addupdate_scatter(
            hist,
            [jnp.right_shift(v, 7), jnp.bitwise_and(v, 127)],
            runlen,
            mask=endm,
        )

    def start_dma(g, j):
      base = base_w + g * BLK
      pltpu.make_async_copy(x_hbm.at[pl.ds(base, BLK)], xbs[j], sxs[j]).start()
      pltpu.make_async_copy(b_hbm.at[pl.ds(base, BLK)], ibs[j], sis[j]).start()

    def wait_dma(j):
      pltpu.make_async_copy(x_hbm.at[pl.ds(0, BLK)], xbs[j], sxs[j]).wait()
      pltpu.make_async_copy(b_hbm.at[pl.ds(0, BLK)], ibs[j], sis[j]).wait()

    def scat_start(j, p):
      pltpu.make_async_copy(xbs[j], acc.at[ibs[j]], sscs[p]).start(add=True)

    def scat_wait(j, p):
      pltpu.make_async_copy(xbs[j], acc.at[ibs[j]], sscs[p]).wait()

    start_dma(0, 0)
    start_dma(1, 1)

    wait_dma(0)
    scat_start(0, 0)
    hist_update(ibs[0])
    start_dma(2, 2)

    wait_dma(1)
    scat_start(1, 1)
    hist_update(ibs[1])

    @pl.loop(0, 20)
    def _(h):
      g6 = 6 * h
      for j in range(6):
        buf = (2 + j) % 3
        p = j % 2
        wait_dma(buf)
        scat_start(buf, p)
        hist_update(ibs[buf])
        scat_wait(j % 3, p)
        start_dma(g6 + 3 + j, j % 3)

    wait_dma(2)
    scat_start(2, 0)
    hist_update(ibs[2])
    scat_wait(0, 0)
    start_dma(123, 0)

    wait_dma(0)
    scat_start(0, 1)
    hist_update(ibs[0])
    scat_wait(1, 1)
    start_dma(124, 1)

    wait_dma(1)
    scat_start(1, 0)
    hist_update(ibs[1])
    scat_wait(2, 0)

    scat_wait(0, 1)
    scat_wait(1, 0)

    plsc.subcore_barrier()
    pltpu.sync_copy(hist, shist.at[i80buf], add=True)
    plsc.subcore_barrier()

    out_base = cid * S_PAD + sid * SROWS
    pltpu.sync_copy(acc.at[pl.ds(sid * SROWS, SROWS)],
                    out_hbm.at[pl.ds(out_base, SROWS)])

    @pl.when(sid == 0)
    def _():
      pltpu.sync_copy(shist, cnt_hbm.at[pl.ds(cid * 80, 80)])

  return k(x, batch_i32, zrow, i80)


def _tc_stats(x):
  def body(x_ref, s_ref, q_ref):
    i = pl.program_id(0)

    @pl.when(i == 0)
    def _():
      s_ref[...] = jnp.zeros_like(s_ref)
      q_ref[...] = jnp.zeros_like(q_ref)

    xb = x_ref[...]
    s_ref[...] += jnp.sum(xb, axis=0, keepdims=True)
    q_ref[...] += jnp.sum(xb * xb, axis=0, keepdims=True)

  return pl.pallas_call(
      body,
      grid=(N // BS,),
      in_specs=[pl.BlockSpec((BS, C), lambda i: (i, 0))],
      out_specs=[
          pl.BlockSpec((1, C), lambda i: (0, 0)),
          pl.BlockSpec((1, C), lambda i: (0, 0)),
      ],
      out_shape=[jax.ShapeDtypeStruct((1, C), jnp.float32)] * 2,
  )(x)


def _finalize(parts, cnts, ssum, ssq, gamma, beta):
  def body(p_ref, c_ref, s_ref, q_ref, g_ref, b_ref, o_ref):
    mean = s_ref[...] / N
    var = q_ref[...] / N - mean * mean
    inv = lax.rsqrt(var + EPS)
    scale = g_ref[...] * inv
    shift = b_ref[...] - mean * scale
    seg = p_ref[0] + p_ref[1]
    cnt = c_ref[0, :, 0:1] + c_ref[1, :, 0:1]
    ok = cnt > 0.0
    cntc = jnp.where(ok, cnt, 1.0)
    o_ref[...] = jnp.where(ok, (seg / cntc) * scale + shift, 0.0)

  return pl.pallas_call(
      body,
      grid=(S_PAD // RB,),
      in_specs=[
          pl.BlockSpec((NC, RB, C), lambda i: (0, i, 0)),
          pl.BlockSpec((NC, RB, C), lambda i: (0, i, 0)),
          pl.BlockSpec((1, C), lambda i: (0, 0)),
          pl.BlockSpec((1, C), lambda i: (0, 0)),
          pl.BlockSpec((1, C), lambda i: (0, 0)),
          pl.BlockSpec((1, C), lambda i: (0, 0)),
      ],
      out_specs=pl.BlockSpec((RB, C), lambda i: (i, 0)),
      out_shape=jax.ShapeDtypeStruct((S_PAD, C), jnp.float32),
  )(parts, cnts, ssum, ssq, gamma, beta)


def kernel(x, batch, gamma, beta):
  batch_i32 = batch.astype(jnp.int32)
  zrow = jnp.zeros((SROWS, C), jnp.float32)
  i80 = jnp.arange(80, dtype=jnp.int32)
  parts, cnth = _sc_segsum(x, batch_i32, zrow, i80)
  cnts = jnp.broadcast_to(
      cnth.reshape(NC, S_PAD)[:, :, None], (NC, S_PAD, C))
  ssum, ssq = _tc_stats(x)
  out = _finalize(
      parts.reshape(NC, S_PAD, C),
      cnts,
      ssum,
      ssq,
      gamma.reshape(1, C),
      beta.reshape(1, C),
  )
  return out[:S]


# speedup vs baseline: 1.2325x; 1.2325x over previous
import dataclasses
import functools

import jax
import jax.numpy as jnp
from jax import lax
from jax.experimental import pallas as pl
from jax.experimental.pallas import tpu as pltpu
from jax.experimental.pallas import tpu_sc as plsc

N = 320000
C = 128
S = 10000
S_PAD = 10240
EPS = 1e-5

NC = 2
NS = 16
NW = NC * NS
ROWS_W = N // NW
BLK = 80
NBLK = ROWS_W // BLK
SROWS = S_PAD // NS

BS = 4000
RB = 2048

_MESH = plsc.VectorSubcoreMesh(core_axis_name="c", subcore_axis_name="s")

_CP = pltpu.CompilerParams()
if "needs_layout_passes" in pltpu.CompilerParams.__dataclass_fields__:
  _CP = dataclasses.replace(_CP, needs_layout_passes=False)


def _sc_segsum(x, batch_i32, zrow, i80):
  @functools.partial(
      pl.kernel,
      out_type=(
          jax.ShapeDtypeStruct((NC * S_PAD, C), jnp.float32),
          jax.ShapeDtypeStruct((NC * 80, C), jnp.float32),
      ),
      mesh=_MESH,
      compiler_params=_CP,
      scratch_types=[
          pltpu.VMEM((BLK, C), jnp.float32),
          pltpu.VMEM((BLK, C), jnp.float32),
          pltpu.VMEM((BLK,), jnp.int32),
          pltpu.VMEM((BLK,), jnp.int32),
          pltpu.VMEM((80, C), jnp.float32),
          pltpu.VMEM((80,), jnp.int32),
          pltpu.VMEM_SHARED((S_PAD, C), jnp.float32),
          pltpu.VMEM_SHARED((80, C), jnp.float32),
          pltpu.SemaphoreType.DMA,
          pltpu.SemaphoreType.DMA,
          pltpu.SemaphoreType.DMA,
          pltpu.SemaphoreType.DMA,
          pltpu.SemaphoreType.DMA,
      ],
  )
  def k(x_hbm, b_hbm, zrow_hbm, i80_hbm, out_hbm, cnt_hbm,
        xb0, xb1, ib0, ib1, hist, i80buf, acc, shist,
        sx0, sx1, si0, si1, ssc):
    cid = lax.axis_index("c")
    sid = lax.axis_index("s")
    wid = cid * NS + sid

    pltpu.sync_copy(zrow_hbm, acc.at[pl.ds(sid * SROWS, SROWS)])
    pltpu.sync_copy(zrow_hbm.at[pl.ds(0, 80)], hist)
    pltpu.sync_copy(i80_hbm, i80buf)

    @pl.when(sid == 0)
    def _():
      pltpu.sync_copy(zrow_hbm.at[pl.ds(0, 80)], shist)

    plsc.subcore_barrier()

    base_w = wid * ROWS_W
    io = lax.iota(jnp.int32, 16)
    pidx = jnp.maximum(io - 1, 0)
    nidx = jnp.minimum(io + 1, 15)

    def take16(v, idx):
      return lax.gather(
          v,
          idx[:, None],
          lax.GatherDimensionNumbers(
              offset_dims=(),
              collapsed_slice_dims=(0,),
              start_index_map=(0,),
          ),
          (1,),
          mode=lax.GatherScatterMode.PROMISE_IN_BOUNDS,
      )

    def hist_update(ib):
      for kk in range(BLK // 16):
        v = ib[pl.ds(16 * kk, 16)]
        prev = take16(v, pidx)
        nxt = take16(v, nidx)
        start_m = (v != prev) | (io == 0)
        s = jnp.where(start_m, io, 0)
        fm = plsc.cummax(s)
        runlen = (io - fm + 1).astype(jnp.float32)
        endm = (v != nxt) | (io == 15)
        plsc.addupdate_scatter(
            hist,
            [jnp.right_shift(v, 7), jnp.bitwise_and(v, 127)],
            runlen,
            mask=endm,
        )

    def start(g, xb, ib, sx, si):
      base = base_w + g * BLK
      pltpu.make_async_copy(x_hbm.at[pl.ds(base, BLK)], xb, sx).start()
      pltpu.make_async_copy(b_hbm.at[pl.ds(base, BLK)], ib, si).start()

    def wait(xb, ib, sx, si):
      pltpu.make_async_copy(x_hbm.at[pl.ds(0, BLK)], xb, sx).wait()
      pltpu.make_async_copy(b_hbm.at[pl.ds(0, BLK)], ib, si).wait()

    def process(xb, ib, sx, si):
      wait(xb, ib, sx, si)
      sc = pltpu.make_async_copy(xb, acc.at[ib], ssc)
      sc.start(add=True)
      hist_update(ib)
      sc.wait()

    start(0, xb0, ib0, sx0, si0)
    start(1, xb1, ib1, sx1, si1)

    @pl.loop(0, (NBLK - 1) // 2)
    def _(h):
      g0 = 2 * h
      process(xb0, ib0, sx0, si0)
      start(g0 + 2, xb0, ib0, sx0, si0)

      process(xb1, ib1, sx1, si1)

      @pl.when(g0 + 3 < NBLK)
      def _():
        start(g0 + 3, xb1, ib1, sx1, si1)

    process(xb0, ib0, sx0, si0)

    plsc.subcore_barrier()
    pltpu.sync_copy(hist, shist.at[i80buf], add=True)
    plsc.subcore_barrier()

    out_base = cid * S_PAD + sid * SROWS
    pltpu.sync_copy(acc.at[pl.ds(sid * SROWS, SROWS)],
                    out_hbm.at[pl.ds(out_base, SROWS)])

    @pl.when(sid == 0)
    def _():
      pltpu.sync_copy(shist, cnt_hbm.at[pl.ds(cid * 80, 80)])

  return k(x, batch_i32, zrow, i80)


def _tc_stats(x):
  def body(x_ref, s_ref, q_ref):
    i = pl.program_id(0)

    @pl.when(i == 0)
    def _():
      s_ref[...] = jnp.zeros_like(s_ref)
      q_ref[...] = jnp.zeros_like(q_ref)

    xb = x_ref[...]
    s_ref[...] += jnp.sum(xb, axis=0, keepdims=True)
    q_ref[...] += jnp.sum(xb * xb, axis=0, keepdims=True)

  return pl.pallas_call(
      body,
      grid=(N // BS,),
      in_specs=[pl.BlockSpec((BS, C), lambda i: (i, 0))],
      out_specs=[
          pl.BlockSpec((1, C), lambda i: (0, 0)),
          pl.BlockSpec((1, C), lambda i: (0, 0)),
      ],
      out_shape=[jax.ShapeDtypeStruct((1, C), jnp.float32)] * 2,
  )(x)


def _finalize(parts, cnts, ssum, ssq, gamma, beta):
  def body(p_ref, c_ref, s_ref, q_ref, g_ref, b_ref, o_ref):
    mean = s_ref[...] / N
    var = q_ref[...] / N - mean * mean
    inv = lax.rsqrt(var + EPS)
    scale = g_ref[...] * inv
    shift = b_ref[...] - mean * scale
    seg = p_ref[0] + p_ref[1]
    cnt = c_ref[0, :, 0:1] + c_ref[1, :, 0:1]
    ok = cnt > 0.0
    cntc = jnp.where(ok, cnt, 1.0)
    o_ref[...] = jnp.where(ok, (seg / cntc) * scale + shift, 0.0)

  return pl.pallas_call(
      body,
      grid=(S_PAD // RB,),
      in_specs=[
          pl.BlockSpec((NC, RB, C), lambda i: (0, i, 0)),
          pl.BlockSpec((NC, RB, C), lambda i: (0, i, 0)),
          pl.BlockSpec((1, C), lambda i: (0, 0)),
          pl.BlockSpec((1, C), lambda i: (0, 0)),
          pl.BlockSpec((1, C), lambda i: (0, 0)),
          pl.BlockSpec((1, C), lambda i: (0, 0)),
      ],
      out_specs=pl.BlockSpec((RB, C), lambda i: (i, 0)),
      out_shape=jax.ShapeDtypeStruct((S_PAD, C), jnp.float32),
  )(parts, cnts, ssum, ssq, gamma, beta)


def kernel(x, batch, gamma, beta):
  batch_i32 = batch.astype(jnp.int32)
  zrow = jnp.zeros((SROWS, C), jnp.float32)
  i80 = jnp.arange(80, dtype=jnp.int32)
  parts, cnth = _sc_segsum(x, batch_i32, zrow, i80)
  cnts = jnp.broadcast_to(
      cnth.reshape(NC, S_PAD)[:, :, None], (NC, S_PAD, C))
  ssum, ssq = _tc_stats(x)
  out = _finalize(
      parts.reshape(NC, S_PAD, C),
      cnts,
      ssum,
      ssq,
      gamma.reshape(1, C),
      beta.reshape(1, C),
  )
  return out[:S]
